# baseline (speedup 1.0000x reference)
import functools

import jax
import jax.numpy as jnp
from jax import lax
from jax.experimental import pallas as pl
from jax.experimental.pallas import tpu as pltpu
from jax.experimental.pallas import tpu_sc as plsc

NUM_CORES = 2
NUM_SUBCORES = 16
NUM_WORKERS = NUM_CORES * NUM_SUBCORES
CHUNK = 128


def _make_kernel(B, CD, D):
    n_chunks = B // CHUNK
    chunks_per_w = n_chunks // NUM_WORKERS
    OUT_D = CD + D

    mesh = plsc.VectorSubcoreMesh(core_axis_name="c", subcore_axis_name="s")

    @functools.partial(
        pl.kernel,
        mesh=mesh,
        out_type=jax.ShapeDtypeStruct((B, OUT_D), jnp.float32),
        scratch_types=[
            pltpu.VMEM((chunks_per_w, CHUNK), jnp.int32),
            pltpu.VMEM((CHUNK, D), jnp.float32),
            pltpu.SemaphoreType.DMA,
        ],
        compiler_params=pltpu.CompilerParams(use_tc_tiling_on_sc=False),
    )
    def k(cond_hbm, idx_hbm, table_hbm, out_hbm, idx_v, rows_v, sem):
        wid = lax.axis_index("s") * NUM_CORES + lax.axis_index("c")
        c0 = wid * chunks_per_w
        pltpu.sync_copy(idx_hbm.at[pl.ds(c0, chunks_per_w)], idx_v)
        for j in range(chunks_per_w):
            row = (c0 + j) * CHUNK
            pltpu.async_copy(table_hbm.at[idx_v.at[j]], rows_v, sem).wait()
            pltpu.sync_copy(
                rows_v, out_hbm.at[pl.ds(row, CHUNK), pl.ds(CD, D)]
            )
            pltpu.sync_copy(
                cond_hbm.at[pl.ds(row, CHUNK)],
                out_hbm.at[pl.ds(row, CHUNK), pl.ds(0, CD)],
            )

    return k


def kernel(cond, task_ids, table):
    B, CD = cond.shape
    _, D = table.shape
    idx2d = task_ids.astype(jnp.int32).reshape(B // CHUNK, CHUNK)
    return _make_kernel(B, CD, D)(cond, idx2d, table)


# speedup vs baseline: 5.8484x; 5.8484x over previous
import functools

import jax
import jax.numpy as jnp
from jax import lax
from jax.experimental import pallas as pl
from jax.experimental.pallas import tpu as pltpu
from jax.experimental.pallas import tpu_sc as plsc

NUM_CORES = 2
NUM_SUBCORES = 16
NUM_WORKERS = NUM_CORES * NUM_SUBCORES
CHUNK = 128


def _make_kernel(B, CD, D):
    n_chunks = B // CHUNK
    chunks_per_w = n_chunks // NUM_WORKERS
    OUT_D = CD + D

    mesh = plsc.VectorSubcoreMesh(core_axis_name="c", subcore_axis_name="s")

    @functools.partial(
        pl.kernel,
        mesh=mesh,
        out_type=jax.ShapeDtypeStruct((B, OUT_D), jnp.float32),
        scratch_types=[
            pltpu.VMEM((chunks_per_w, CHUNK), jnp.int32),
            pltpu.VMEM((2, CHUNK, OUT_D), jnp.float32),
            pltpu.VMEM((2, CHUNK, D), jnp.float32),
            pltpu.SemaphoreType.DMA,
            pltpu.SemaphoreType.DMA,
            pltpu.SemaphoreType.DMA,
            pltpu.SemaphoreType.DMA,
            pltpu.SemaphoreType.DMA,
            pltpu.SemaphoreType.DMA,
        ],
        compiler_params=pltpu.CompilerParams(use_tc_tiling_on_sc=False),
    )
    def k(cond_hbm, idx_hbm, table_hbm, out_hbm, idx_v, out_v, emb_v,
          sg0, sg1, sc0, sc1, so0, so1):
        sg = (sg0, sg1)
        sc = (sc0, sc1)
        so = (so0, so1)
        wid = lax.axis_index("s") * NUM_CORES + lax.axis_index("c")
        c0 = wid * chunks_per_w
        pltpu.sync_copy(idx_hbm.at[pl.ds(c0, chunks_per_w)], idx_v)

        in_cps = [None] * chunks_per_w
        out_cps = [None] * chunks_per_w

        def issue(j):
            slot = j % 2
            row = (c0 + j) * CHUNK
            g = pltpu.async_copy(
                table_hbm.at[idx_v.at[j]],
                emb_v.at[slot],
                sg[slot],
            )
            c = pltpu.async_copy(
                cond_hbm.at[pl.ds(row, CHUNK)],
                out_v.at[slot, :, pl.ds(0, CD)],
                sc[slot],
            )
            in_cps[j] = (g, c)

        def finish(j):
            slot = j % 2
            row = (c0 + j) * CHUNK
            g, c = in_cps[j]
            g.wait()
            c.wait()

            def merge(r, carry):
                for gidx in range(D // 16):
                    out_v[slot, r, pl.ds(CD + gidx * 16, 16)] = (
                        emb_v[slot, r, pl.ds(gidx * 16, 16)]
                    )
                return carry

            lax.fori_loop(0, CHUNK, merge, 0)
            out_cps[j] = pltpu.async_copy(
                out_v.at[slot], out_hbm.at[pl.ds(row, CHUNK)], so[slot]
            )

        for j in range(chunks_per_w):
            if j >= 2:
                out_cps[j - 2].wait()
            issue(j)
            if j >= 1:
                finish(j - 1)
        finish(chunks_per_w - 1)
        if chunks_per_w >= 2:
            out_cps[chunks_per_w - 2].wait()
        out_cps[chunks_per_w - 1].wait()

    return k


def kernel(cond, task_ids, table):
    B, CD = cond.shape
    _, D = table.shape
    idx2d = task_ids.astype(jnp.int32).reshape(B // CHUNK, CHUNK)
    return _make_kernel(B, CD, D)(cond, idx2d, table)


# speedup vs baseline: 7.1424x; 1.2212x over previous
import functools

import jax
import jax.numpy as jnp
from jax import lax
from jax.experimental import pallas as pl
from jax.experimental.pallas import tpu as pltpu
from jax.experimental.pallas import tpu_sc as plsc

NUM_CORES = 2
NUM_SUBCORES = 16
NUM_WORKERS = NUM_CORES * NUM_SUBCORES
CHUNK = 128
TPAD = 128
BR = 512


def _make_gather(B, V):
    n_chunks = B // CHUNK
    chunks_per_w = n_chunks // NUM_WORKERS

    mesh = plsc.VectorSubcoreMesh(core_axis_name="c", subcore_axis_name="s")

    @functools.partial(
        pl.kernel,
        mesh=mesh,
        out_type=jax.ShapeDtypeStruct((B, TPAD), jnp.float32),
        scratch_types=[
            pltpu.VMEM((chunks_per_w, CHUNK), jnp.int32),
            pltpu.VMEM((2, CHUNK, TPAD), jnp.float32),
            pltpu.SemaphoreType.DMA,
            pltpu.SemaphoreType.DMA,
            pltpu.SemaphoreType.DMA,
            pltpu.SemaphoreType.DMA,
        ],
    )
    def k(idx_hbm, table_hbm, emb_hbm, idx_v, emb_v, sg0, sg1, so0, so1):
        sg = (sg0, sg1)
        so = (so0, so1)
        wid = lax.axis_index("s") * NUM_CORES + lax.axis_index("c")
        c0 = wid * chunks_per_w
        for j in range(chunks_per_w):
            pltpu.sync_copy(
                idx_hbm.at[pl.ds((c0 + j) * CHUNK, CHUNK)], idx_v.at[j]
            )

        in_cps = [None] * chunks_per_w
        out_cps = [None] * chunks_per_w

        def issue(j):
            slot = j % 2
            in_cps[j] = pltpu.async_copy(
                table_hbm.at[idx_v.at[j]], emb_v.at[slot], sg[slot]
            )

        def finish(j):
            slot = j % 2
            row = (c0 + j) * CHUNK
            in_cps[j].wait()
            out_cps[j] = pltpu.async_copy(
                emb_v.at[slot], emb_hbm.at[pl.ds(row, CHUNK)], so[slot]
            )

        for j in range(chunks_per_w):
            if j >= 2:
                out_cps[j - 2].wait()
            issue(j)
            if j >= 1:
                finish(j - 1)
        finish(chunks_per_w - 1)
        if chunks_per_w >= 2:
            out_cps[chunks_per_w - 2].wait()
        out_cps[chunks_per_w - 1].wait()

    return k


def _concat_body(cond_ref, emb_ref, out_ref):
    D = out_ref.shape[1] - cond_ref.shape[1]
    out_ref[:, : cond_ref.shape[1]] = cond_ref[...]
    out_ref[:, cond_ref.shape[1]:] = emb_ref[:, :D]


def _make_concat(B, CD, D):
    OUT_D = CD + D
    return pl.pallas_call(
        _concat_body,
        grid=(B // BR,),
        in_specs=[
            pl.BlockSpec((BR, CD), lambda i: (i, 0)),
            pl.BlockSpec((BR, TPAD), lambda i: (i, 0)),
        ],
        out_specs=pl.BlockSpec((BR, OUT_D), lambda i: (i, 0)),
        out_shape=jax.ShapeDtypeStruct((B, OUT_D), jnp.float32),
        compiler_params=pltpu.CompilerParams(
            dimension_semantics=("arbitrary",),
        ),
    )


def kernel(cond, task_ids, table):
    B, CD = cond.shape
    V, D = table.shape
    idx = task_ids.astype(jnp.int32)
    table_p = jnp.pad(table, ((0, 0), (0, TPAD - D)))
    emb = _make_gather(B, V)(idx, table_p)
    return _make_concat(B, CD, D)(cond, emb)


# speedup vs baseline: 8.0610x; 1.1286x over previous
import functools

import jax
import jax.numpy as jnp
from jax import lax
from jax.experimental import pallas as pl
from jax.experimental.pallas import tpu as pltpu
from jax.experimental.pallas import tpu_sc as plsc

NUM_CORES = 2
NUM_SUBCORES = 16
NUM_WORKERS = NUM_CORES * NUM_SUBCORES
CHUNK = 128
TPAD = 128
BR = 1024


def _make_gather(B, V):
    n_chunks = B // CHUNK
    chunks_per_w = n_chunks // NUM_WORKERS

    mesh = plsc.VectorSubcoreMesh(core_axis_name="c", subcore_axis_name="s")

    @functools.partial(
        pl.kernel,
        mesh=mesh,
        out_type=jax.ShapeDtypeStruct((B, TPAD), jnp.float32),
        scratch_types=[
            pltpu.VMEM((chunks_per_w, CHUNK), jnp.int32),
            pltpu.VMEM((2, CHUNK, TPAD), jnp.float32),
            pltpu.SemaphoreType.DMA,
            pltpu.SemaphoreType.DMA,
            pltpu.SemaphoreType.DMA,
            pltpu.SemaphoreType.DMA,
        ],
    )
    def k(idx_hbm, table_hbm, emb_hbm, idx_v, emb_v, sg0, sg1, so0, so1):
        sg = (sg0, sg1)
        so = (so0, so1)
        wid = lax.axis_index("s") * NUM_CORES + lax.axis_index("c")
        c0 = wid * chunks_per_w
        for j in range(chunks_per_w):
            pltpu.sync_copy(
                idx_hbm.at[pl.ds((c0 + j) * CHUNK, CHUNK)], idx_v.at[j]
            )

        in_cps = [None] * chunks_per_w
        out_cps = [None] * chunks_per_w

        def issue(j):
            slot = j % 2
            in_cps[j] = pltpu.async_copy(
                table_hbm.at[idx_v.at[j]], emb_v.at[slot], sg[slot]
            )

        def finish(j):
            slot = j % 2
            row = (c0 + j) * CHUNK
            in_cps[j].wait()
            out_cps[j] = pltpu.async_copy(
                emb_v.at[slot], emb_hbm.at[pl.ds(row, CHUNK)], so[slot]
            )

        for j in range(chunks_per_w):
            if j >= 2:
                out_cps[j - 2].wait()
            issue(j)
            if j >= 1:
                finish(j - 1)
        finish(chunks_per_w - 1)
        if chunks_per_w >= 2:
            out_cps[chunks_per_w - 2].wait()
        out_cps[chunks_per_w - 1].wait()

    return k


def _concat_body(cond_ref, emb_ref, out_ref):
    cd = cond_ref.shape[1]
    d = out_ref.shape[1] - cd
    out_ref[:, :cd] = cond_ref[...]
    out_ref[:, cd:] = emb_ref[:, :d]


def _make_concat(B, CD, D):
    OUT_D = CD + D
    return pl.pallas_call(
        _concat_body,
        grid=(B // BR,),
        in_specs=[
            pl.BlockSpec((BR, CD), lambda i: (i, 0)),
            pl.BlockSpec((BR, TPAD), lambda i: (i, 0)),
        ],
        out_specs=pl.BlockSpec((BR, OUT_D), lambda i: (i, 0)),
        out_shape=jax.ShapeDtypeStruct((B, OUT_D), jnp.float32),
        compiler_params=pltpu.CompilerParams(
            dimension_semantics=("arbitrary",),
        ),
    )


def kernel(cond, task_ids, table):
    B, CD = cond.shape
    V, D = table.shape
    idx = task_ids.astype(jnp.int32)
    table_p = jnp.pad(table, ((0, 0), (0, TPAD - D)))
    emb = _make_gather(B, V)(idx, table_p)
    return _make_concat(B, CD, D)(cond, emb)


# speedup vs baseline: 8.3189x; 1.0320x over previous
import functools

import jax
import jax.numpy as jnp
from jax import lax
from jax.experimental import pallas as pl
from jax.experimental.pallas import tpu as pltpu
from jax.experimental.pallas import tpu_sc as plsc

NUM_CORES = 2
NUM_SUBCORES = 16
NUM_WORKERS = NUM_CORES * NUM_SUBCORES
CHUNK = 128
TPAD = 128
BR = 2048


def _make_gather(B, V):
    n_chunks = B // CHUNK
    chunks_per_w = n_chunks // NUM_WORKERS

    mesh = plsc.VectorSubcoreMesh(core_axis_name="c", subcore_axis_name="s")

    @functools.partial(
        pl.kernel,
        mesh=mesh,
        out_type=jax.ShapeDtypeStruct((B, TPAD), jnp.float32),
        scratch_types=[
            pltpu.VMEM((chunks_per_w, CHUNK), jnp.int32),
            pltpu.VMEM((2, CHUNK, TPAD), jnp.float32),
            pltpu.SemaphoreType.DMA,
            pltpu.SemaphoreType.DMA,
            pltpu.SemaphoreType.DMA,
            pltpu.SemaphoreType.DMA,
        ],
    )
    def k(idx_hbm, table_hbm, emb_hbm, idx_v, emb_v, sg0, sg1, so0, so1):
        sg = (sg0, sg1)
        so = (so0, so1)
        wid = lax.axis_index("s") * NUM_CORES + lax.axis_index("c")
        c0 = wid * chunks_per_w
        for j in range(chunks_per_w):
            pltpu.sync_copy(
                idx_hbm.at[pl.ds((c0 + j) * CHUNK, CHUNK)], idx_v.at[j]
            )

        in_cps = [None] * chunks_per_w
        out_cps = [None] * chunks_per_w

        def issue(j):
            slot = j % 2
            in_cps[j] = pltpu.async_copy(
                table_hbm.at[idx_v.at[j]], emb_v.at[slot], sg[slot]
            )

        def finish(j):
            slot = j % 2
            row = (c0 + j) * CHUNK
            in_cps[j].wait()
            out_cps[j] = pltpu.async_copy(
                emb_v.at[slot], emb_hbm.at[pl.ds(row, CHUNK)], so[slot]
            )

        for j in range(chunks_per_w):
            if j >= 2:
                out_cps[j - 2].wait()
            issue(j)
            if j >= 1:
                finish(j - 1)
        finish(chunks_per_w - 1)
        if chunks_per_w >= 2:
            out_cps[chunks_per_w - 2].wait()
        out_cps[chunks_per_w - 1].wait()

    return k


def _concat_body(cond_ref, emb_ref, out_ref):
    cd = cond_ref.shape[1]
    d = out_ref.shape[1] - cd
    out_ref[:, :cd] = cond_ref[...]
    out_ref[:, cd:] = emb_ref[:, :d]


def _make_concat(B, CD, D):
    OUT_D = CD + D
    return pl.pallas_call(
        _concat_body,
        grid=(B // BR,),
        in_specs=[
            pl.BlockSpec((BR, CD), lambda i: (i, 0)),
            pl.BlockSpec((BR, TPAD), lambda i: (i, 0)),
        ],
        out_specs=pl.BlockSpec((BR, OUT_D), lambda i: (i, 0)),
        out_shape=jax.ShapeDtypeStruct((B, OUT_D), jnp.float32),
        compiler_params=pltpu.CompilerParams(
            dimension_semantics=("arbitrary",),
        ),
    )


def kernel(cond, task_ids, table):
    B, CD = cond.shape
    V, D = table.shape
    idx = task_ids.astype(jnp.int32)
    table_p = jnp.pad(table, ((0, 0), (0, TPAD - D)))
    emb = _make_gather(B, V)(idx, table_p)
    return _make_concat(B, CD, D)(cond, emb)


# speedup vs baseline: 8.4725x; 1.0185x over previous
import functools

import jax
import jax.numpy as jnp
from jax import lax
from jax.experimental import pallas as pl
from jax.experimental.pallas import tpu as pltpu
from jax.experimental.pallas import tpu_sc as plsc

NUM_CORES = 2
NUM_SUBCORES = 16
NUM_WORKERS = NUM_CORES * NUM_SUBCORES
CHUNK = 128
TPAD = 128
BR = 4096


def _make_gather(B, V):
    n_chunks = B // CHUNK
    chunks_per_w = n_chunks // NUM_WORKERS

    mesh = plsc.VectorSubcoreMesh(core_axis_name="c", subcore_axis_name="s")

    @functools.partial(
        pl.kernel,
        mesh=mesh,
        out_type=jax.ShapeDtypeStruct((B, TPAD), jnp.float32),
        scratch_types=[
            pltpu.VMEM((chunks_per_w, CHUNK), jnp.int32),
            pltpu.VMEM((2, CHUNK, TPAD), jnp.float32),
            pltpu.SemaphoreType.DMA,
            pltpu.SemaphoreType.DMA,
            pltpu.SemaphoreType.DMA,
            pltpu.SemaphoreType.DMA,
        ],
    )
    def k(idx_hbm, table_hbm, emb_hbm, idx_v, emb_v, sg0, sg1, so0, so1):
        sg = (sg0, sg1)
        so = (so0, so1)
        wid = lax.axis_index("s") * NUM_CORES + lax.axis_index("c")
        c0 = wid * chunks_per_w
        for j in range(chunks_per_w):
            pltpu.sync_copy(
                idx_hbm.at[pl.ds((c0 + j) * CHUNK, CHUNK)], idx_v.at[j]
            )

        in_cps = [None] * chunks_per_w
        out_cps = [None] * chunks_per_w

        def issue(j):
            slot = j % 2
            in_cps[j] = pltpu.async_copy(
                table_hbm.at[idx_v.at[j]], emb_v.at[slot], sg[slot]
            )

        def finish(j):
            slot = j % 2
            row = (c0 + j) * CHUNK
            in_cps[j].wait()
            out_cps[j] = pltpu.async_copy(
                emb_v.at[slot], emb_hbm.at[pl.ds(row, CHUNK)], so[slot]
            )

        for j in range(chunks_per_w):
            if j >= 2:
                out_cps[j - 2].wait()
            issue(j)
            if j >= 1:
                finish(j - 1)
        finish(chunks_per_w - 1)
        if chunks_per_w >= 2:
            out_cps[chunks_per_w - 2].wait()
        out_cps[chunks_per_w - 1].wait()

    return k


def _concat_body(cond_ref, emb_ref, out_ref):
    cd = cond_ref.shape[1]
    d = out_ref.shape[1] - cd
    out_ref[:, :cd] = cond_ref[...]
    out_ref[:, cd:] = emb_ref[:, :d]


def _make_concat(B, CD, D):
    OUT_D = CD + D
    return pl.pallas_call(
        _concat_body,
        grid=(B // BR,),
        in_specs=[
            pl.BlockSpec((BR, CD), lambda i: (i, 0)),
            pl.BlockSpec((BR, TPAD), lambda i: (i, 0)),
        ],
        out_specs=pl.BlockSpec((BR, OUT_D), lambda i: (i, 0)),
        out_shape=jax.ShapeDtypeStruct((B, OUT_D), jnp.float32),
        compiler_params=pltpu.CompilerParams(
            dimension_semantics=("arbitrary",),
        ),
    )


def kernel(cond, task_ids, table):
    B, CD = cond.shape
    V, D = table.shape
    idx = task_ids.astype(jnp.int32)
    table_p = jnp.pad(table, ((0, 0), (0, TPAD - D)))
    emb = _make_gather(B, V)(idx, table_p)
    return _make_concat(B, CD, D)(cond, emb)


# speedup vs baseline: 8.7177x; 1.0289x over previous
import functools

import jax
import jax.numpy as jnp
from jax import lax
from jax.experimental import pallas as pl
from jax.experimental.pallas import tpu as pltpu
from jax.experimental.pallas import tpu_sc as plsc

NUM_CORES = 2
NUM_SUBCORES = 16
NUM_WORKERS = NUM_CORES * NUM_SUBCORES
CHUNK = 128
TPAD = 128
BR = 4096


def _make_gather(B, V):
    n_chunks = B // CHUNK
    chunks_per_w = n_chunks // NUM_WORKERS

    mesh = plsc.VectorSubcoreMesh(core_axis_name="c", subcore_axis_name="s")

    @functools.partial(
        pl.kernel,
        mesh=mesh,
        out_type=jax.ShapeDtypeStruct((B, TPAD), jnp.float32),
        scratch_types=[
            pltpu.VMEM((chunks_per_w, CHUNK), jnp.int32),
            pltpu.VMEM((2, CHUNK, TPAD), jnp.float32),
            pltpu.SemaphoreType.DMA,
            pltpu.SemaphoreType.DMA,
            pltpu.SemaphoreType.DMA,
            pltpu.SemaphoreType.DMA,
        ],
    )
    def k(idx_hbm, table_hbm, emb_hbm, idx_v, emb_v, sg0, sg1, so0, so1):
        sg = (sg0, sg1)
        so = (so0, so1)
        wid = lax.axis_index("s") * NUM_CORES + lax.axis_index("c")
        c0 = wid * chunks_per_w
        for j in range(chunks_per_w):
            pltpu.sync_copy(
                idx_hbm.at[pl.ds((c0 + j) * CHUNK, CHUNK)], idx_v.at[j]
            )

        in_cps = [None] * chunks_per_w
        out_cps = [None] * chunks_per_w

        def issue(j):
            slot = j % 2
            in_cps[j] = pltpu.async_copy(
                table_hbm.at[idx_v.at[j]], emb_v.at[slot], sg[slot]
            )

        def finish(j):
            slot = j % 2
            row = (c0 + j) * CHUNK
            in_cps[j].wait()
            out_cps[j] = pltpu.async_copy(
                emb_v.at[slot], emb_hbm.at[pl.ds(row, CHUNK)], so[slot]
            )

        for j in range(chunks_per_w):
            if j >= 2:
                out_cps[j - 2].wait()
            issue(j)
            if j >= 1:
                finish(j - 1)
        finish(chunks_per_w - 1)
        if chunks_per_w >= 2:
            out_cps[chunks_per_w - 2].wait()
        out_cps[chunks_per_w - 1].wait()

    return k


def _cond_copy_body(cond_ref, out_ref):
    out_ref[...] = cond_ref[...]


def _make_cond_copy(B, CD, OUT_D):
    return pl.pallas_call(
        _cond_copy_body,
        grid=(B // BR,),
        in_specs=[pl.BlockSpec((BR, CD), lambda i: (i, 0))],
        out_specs=pl.BlockSpec((BR, CD), lambda i: (i, 0)),
        out_shape=jax.ShapeDtypeStruct((B, OUT_D), jnp.float32),
        compiler_params=pltpu.CompilerParams(
            dimension_semantics=("arbitrary",),
        ),
    )


def _emb_insert_body(emb_ref, acc_ref, out_ref):
    del acc_ref
    out_ref[...] = emb_ref[...]


def _make_emb_insert(B, CD, OUT_D):
    assert CD % TPAD == 0 and CD // TPAD == 2
    return pl.pallas_call(
        _emb_insert_body,
        grid=(B // BR,),
        in_specs=[
            pl.BlockSpec((BR, TPAD), lambda i: (i, 0)),
            pl.BlockSpec(memory_space=pl.ANY),
        ],
        out_specs=pl.BlockSpec((BR, TPAD), lambda i: (i, 2)),
        out_shape=jax.ShapeDtypeStruct((B, OUT_D), jnp.float32),
        input_output_aliases={1: 0},
        compiler_params=pltpu.CompilerParams(
            dimension_semantics=("arbitrary",),
        ),
    )


def kernel(cond, task_ids, table):
    B, CD = cond.shape
    V, D = table.shape
    idx = task_ids.astype(jnp.int32)
    table_p = jnp.pad(table, ((0, 0), (0, TPAD - D)))
    emb = _make_gather(B, V)(idx, table_p)
    acc = _make_cond_copy(B, CD, CD + D)(cond)
    return _make_emb_insert(B, CD, CD + D)(emb, acc)


# speedup vs baseline: 13.1315x; 1.5063x over previous
import functools

import jax
import jax.numpy as jnp
from jax import lax
from jax.experimental import pallas as pl
from jax.experimental.pallas import tpu as pltpu
from jax.experimental.pallas import tpu_sc as plsc

NUM_CORES = 2
NUM_SUBCORES = 16
NUM_WORKERS = NUM_CORES * NUM_SUBCORES
CHUNK = 128
TPAD = 128
BR = 4096


def _make_gather(B, V):
    n_chunks = B // CHUNK
    chunks_per_w = n_chunks // NUM_WORKERS

    mesh = plsc.VectorSubcoreMesh(core_axis_name="c", subcore_axis_name="s")

    @functools.partial(
        pl.kernel,
        mesh=mesh,
        out_type=jax.ShapeDtypeStruct((B, TPAD), jnp.float32),
        scratch_types=[
            pltpu.VMEM((chunks_per_w, CHUNK), jnp.int32),
            pltpu.VMEM((2, CHUNK, TPAD), jnp.float32),
            pltpu.SemaphoreType.DMA,
            pltpu.SemaphoreType.DMA,
            pltpu.SemaphoreType.DMA,
            pltpu.SemaphoreType.DMA,
        ],
    )
    def k(idx_hbm, table_hbm, emb_hbm, idx_v, emb_v, sg0, sg1, so0, so1):
        sg = (sg0, sg1)
        so = (so0, so1)
        wid = lax.axis_index("s") * NUM_CORES + lax.axis_index("c")
        c0 = wid * chunks_per_w
        for j in range(chunks_per_w):
            pltpu.sync_copy(
                idx_hbm.at[pl.ds((c0 + j) * CHUNK, CHUNK)], idx_v.at[j]
            )

        in_cps = [None] * chunks_per_w
        out_cps = [None] * chunks_per_w

        def issue(j):
            slot = j % 2
            in_cps[j] = pltpu.async_copy(
                table_hbm.at[idx_v.at[j]], emb_v.at[slot], sg[slot]
            )

        def finish(j):
            slot = j % 2
            row = (c0 + j) * CHUNK
            in_cps[j].wait()
            out_cps[j] = pltpu.async_copy(
                emb_v.at[slot], emb_hbm.at[pl.ds(row, CHUNK)], so[slot]
            )

        for j in range(chunks_per_w):
            if j >= 2:
                out_cps[j - 2].wait()
            issue(j)
            if j >= 1:
                finish(j - 1)
        finish(chunks_per_w - 1)
        if chunks_per_w >= 2:
            out_cps[chunks_per_w - 2].wait()
        out_cps[chunks_per_w - 1].wait()

    return k


def _cond_t_body(cond_ref, out_ref):
    out_ref[...] = cond_ref[...].T


def _make_cond_t(B, CD, OUT_D):
    return pl.pallas_call(
        _cond_t_body,
        grid=(B // BR,),
        in_specs=[pl.BlockSpec((BR, CD), lambda i: (i, 0))],
        out_specs=pl.BlockSpec((CD, BR), lambda i: (0, i)),
        out_shape=jax.ShapeDtypeStruct((OUT_D, B), jnp.float32),
        compiler_params=pltpu.CompilerParams(
            dimension_semantics=("arbitrary",),
        ),
    )


def _emb_insert_body(emb_ref, acc_ref, out_ref):
    del acc_ref
    out_ref[...] = emb_ref[:, : out_ref.shape[0]].T


def _make_emb_insert(B, CD, D, OUT_D):
    assert CD % D == 0
    return pl.pallas_call(
        _emb_insert_body,
        grid=(B // BR,),
        in_specs=[
            pl.BlockSpec((BR, TPAD), lambda i: (i, 0)),
            pl.BlockSpec(memory_space=pl.ANY),
        ],
        out_specs=pl.BlockSpec((D, BR), lambda i: (CD // D, i)),
        out_shape=jax.ShapeDtypeStruct((OUT_D, B), jnp.float32),
        input_output_aliases={1: 0},
        compiler_params=pltpu.CompilerParams(
            dimension_semantics=("arbitrary",),
        ),
    )


def kernel(cond, task_ids, table):
    B, CD = cond.shape
    V, D = table.shape
    idx = task_ids.astype(jnp.int32)
    table_p = jnp.pad(table, ((0, 0), (0, TPAD - D)))
    emb = _make_gather(B, V)(idx, table_p)
    acc_t = _make_cond_t(B, CD, CD + D)(cond)
    out_t = _make_emb_insert(B, CD, D, CD + D)(emb, acc_t)
    return out_t.T
